# Initial kernel scaffold; baseline (speedup 1.0000x reference)
import jax
import jax.numpy as jnp
from jax.experimental import pallas as pl


def kernel(point_features, point_coords, W0, b0, We1, be1, g1, bt1, W2, b2, We3, be3, g3, bt3, Wout, bout):
    raise NotImplementedError("write your pallas kernel here")


# speedup vs baseline: 4.7814x; 4.7814x over previous
import functools

import jax
import jax.numpy as jnp
from jax import lax
from jax.experimental import pallas as pl
from jax.experimental.pallas import tpu as pltpu
from jax.experimental.pallas import tpu_sc as plsc

N = 10000
K = 16
NPAD = 10240
T_KNN = 256
BIG = 3.0e38


def _knn_body(posr_ref, post_ref, idx_ref):
    t = posr_ref.shape[0]
    npad = post_ref.shape[1]
    d = None
    for c in range(3):
        a = posr_ref[:, c:c + 1]
        b = post_ref[c:c + 1, :]
        diff = a - b
        sq = diff * diff
        d = sq if d is None else d + sq
    cid = lax.broadcasted_iota(jnp.int32, (t, npad), 1)
    cols = []
    for _ in range(K):
        m = jnp.min(d, axis=1, keepdims=True)
        cand = jnp.where(d == m, cid, npad)
        j = jnp.min(cand, axis=1, keepdims=True)
        cols.append(j)
        d = jnp.where(cid == j, BIG, d)
    idx_ref[...] = jnp.concatenate(cols, axis=1)


def _knn(posr, post):
    grid = (NPAD // T_KNN,)
    return pl.pallas_call(
        _knn_body,
        grid=grid,
        in_specs=[
            pl.BlockSpec((T_KNN, 4), lambda i: (i, 0)),
            pl.BlockSpec((8, NPAD), lambda i: (0, 0)),
        ],
        out_specs=pl.BlockSpec((T_KNN, K), lambda i: (i, 0)),
        out_shape=jax.ShapeDtypeStruct((NPAD, K), jnp.int32),
    )(posr, post)


def _mm1_body(x_ref, w0_ref, b0_ref, we_ref, be_ref, f1_ref, u_ref, v_ref):
    f1 = jnp.dot(x_ref[...], w0_ref[...], preferred_element_type=jnp.float32)
    f1 = f1 + b0_ref[0:1, :]
    c = w0_ref.shape[1]
    wa = we_ref[0:c, :]
    wb = we_ref[c:2 * c, :]
    u_ref[...] = jnp.dot(f1, wa - wb, preferred_element_type=jnp.float32) + be_ref[0:1, :]
    v_ref[...] = jnp.dot(f1, wb, preferred_element_type=jnp.float32)
    f1_ref[...] = f1


def _mm1(x, w0, b0r, we1, be1r):
    tr = 1000
    grid = (N // tr,)
    return pl.pallas_call(
        _mm1_body,
        grid=grid,
        in_specs=[
            pl.BlockSpec((tr, 64), lambda i: (i, 0)),
            pl.BlockSpec((64, 64), lambda i: (0, 0)),
            pl.BlockSpec((8, 64), lambda i: (0, 0)),
            pl.BlockSpec((128, 64), lambda i: (0, 0)),
            pl.BlockSpec((8, 64), lambda i: (0, 0)),
        ],
        out_specs=[
            pl.BlockSpec((tr, 64), lambda i: (i, 0)),
            pl.BlockSpec((tr, 64), lambda i: (i, 0)),
            pl.BlockSpec((tr, 64), lambda i: (i, 0)),
        ],
        out_shape=[jax.ShapeDtypeStruct((N, 64), jnp.float32)] * 3,
    )(x, w0, b0r, we1, be1r)


def _sc_gather_reduce(v_pad, idx_flat, c_dim):
    c_tab = v_pad.shape[1]
    nw = 32
    bpw = NPAD // nw
    nb = bpw // 8
    mesh = plsc.VectorSubcoreMesh(core_axis_name="c", subcore_axis_name="s")

    @functools.partial(
        pl.kernel,
        mesh=mesh,
        out_type=[jax.ShapeDtypeStruct((NPAD, c_dim), jnp.float32)] * 4,
        scratch_types=[
            pltpu.VMEM((128,), jnp.int32),
            pltpu.VMEM((128, c_tab), jnp.float32),
            pltpu.VMEM((8, c_dim), jnp.float32),
            pltpu.VMEM((8, c_dim), jnp.float32),
            pltpu.VMEM((8, c_dim), jnp.float32),
            pltpu.VMEM((8, c_dim), jnp.float32),
            pltpu.SemaphoreType.DMA,
        ],
    )
    def k(v_hbm, idx_hbm, s_hbm, q_hbm, mx_hbm, mn_hbm,
          idx_v, rows_v, s_v, q_v, mx_v, mn_v, sem):
        wid = lax.axis_index("s") * 2 + lax.axis_index("c")
        base = wid * bpw

        def batch(bi, carry):
            n0 = base + bi * 8
            pltpu.sync_copy(idx_hbm.at[pl.ds(n0 * K, 8 * K)], idx_v)
            pltpu.async_copy(v_hbm.at[idx_v], rows_v, sem).wait()

            def node(n, carry2):
                r0 = n * K
                for ch in range(c_dim // 16):
                    sl = pl.ds(ch * 16, 16)
                    x0 = rows_v[r0, sl]
                    sa = x0
                    qa = x0 * x0
                    mxa = x0
                    mna = x0
                    for r in range(1, K):
                        xr = rows_v[r0 + r, sl]
                        sa = sa + xr
                        qa = qa + xr * xr
                        mxa = jnp.maximum(mxa, xr)
                        mna = jnp.minimum(mna, xr)
                    s_v[n, sl] = sa
                    q_v[n, sl] = qa
                    mx_v[n, sl] = mxa
                    mn_v[n, sl] = mna
                return carry2

            lax.fori_loop(0, 8, node, 0)
            pltpu.sync_copy(s_v, s_hbm.at[pl.ds(n0, 8), :])
            pltpu.sync_copy(q_v, q_hbm.at[pl.ds(n0, 8), :])
            pltpu.sync_copy(mx_v, mx_hbm.at[pl.ds(n0, 8), :])
            pltpu.sync_copy(mn_v, mn_hbm.at[pl.ds(n0, 8), :])
            return carry

        lax.fori_loop(0, nb, batch, 0)

    return k(v_pad, idx_flat)


def _stats_body(u_ref, s_ref, q_ref, g_ref, bt_ref, st_ref):
    u = u_ref[...]
    s = s_ref[...]
    q = q_ref[...]
    su = jnp.sum(u, axis=0, keepdims=True)
    su2 = jnp.sum(u * u, axis=0, keepdims=True)
    sus = jnp.sum(u * s, axis=0, keepdims=True)
    ss = jnp.sum(s, axis=0, keepdims=True)
    sq = jnp.sum(q, axis=0, keepdims=True)
    denom = jnp.float32(N * K)
    mu = (K * su + ss) / denom
    eh2 = (K * su2 + 2.0 * sus + sq) / denom
    var = eh2 - mu * mu
    sc = g_ref[0:1, :] * lax.rsqrt(var + 1e-5)
    tc = bt_ref[0:1, :] - mu * sc
    st_ref[...] = jnp.concatenate([sc, tc, sc, tc, sc, tc, sc, tc], axis=0)


def _stats(u, s, q, gr, btr):
    c = u.shape[1]
    return pl.pallas_call(
        _stats_body,
        grid=(1,),
        in_specs=[
            pl.BlockSpec((N, c), lambda i: (0, 0)),
            pl.BlockSpec((N, c), lambda i: (0, 0)),
            pl.BlockSpec((N, c), lambda i: (0, 0)),
            pl.BlockSpec((8, c), lambda i: (0, 0)),
            pl.BlockSpec((8, c), lambda i: (0, 0)),
        ],
        out_specs=pl.BlockSpec((8, c), lambda i: (0, 0)),
        out_shape=jax.ShapeDtypeStruct((8, c), jnp.float32),
    )(u, s, q, gr, btr)


def _mid_body(f1_ref, u1_ref, mx_ref, mn_ref, st_ref, w2_ref, b2_ref,
              we3_ref, be3_ref, f2_ref, f3_ref, u3_ref, v3_ref):
    s = st_ref[0:1, :]
    t = st_ref[1:2, :]
    hx = u1_ref[...] + jnp.where(s >= 0, mx_ref[...], mn_ref[...])
    f2 = jnp.maximum(s * hx + t, 0.0) + f1_ref[...]
    f3 = jnp.dot(f2, w2_ref[...], preferred_element_type=jnp.float32) + b2_ref[0:1, :]
    c = w2_ref.shape[1]
    wa = we3_ref[0:c, :]
    wb = we3_ref[c:2 * c, :]
    u3_ref[...] = jnp.dot(f3, wa - wb, preferred_element_type=jnp.float32) + be3_ref[0:1, :]
    v3_ref[...] = jnp.dot(f3, wb, preferred_element_type=jnp.float32)
    f2_ref[...] = f2
    f3_ref[...] = f3


def _mid(f1, u1, mx1, mn1, st1, w2, b2r, we3, be3r):
    tr = 1000
    grid = (N // tr,)
    return pl.pallas_call(
        _mid_body,
        grid=grid,
        in_specs=[
            pl.BlockSpec((tr, 64), lambda i: (i, 0)),
            pl.BlockSpec((tr, 64), lambda i: (i, 0)),
            pl.BlockSpec((tr, 64), lambda i: (i, 0)),
            pl.BlockSpec((tr, 64), lambda i: (i, 0)),
            pl.BlockSpec((8, 64), lambda i: (0, 0)),
            pl.BlockSpec((64, 128), lambda i: (0, 0)),
            pl.BlockSpec((8, 128), lambda i: (0, 0)),
            pl.BlockSpec((256, 128), lambda i: (0, 0)),
            pl.BlockSpec((8, 128), lambda i: (0, 0)),
        ],
        out_specs=[
            pl.BlockSpec((tr, 64), lambda i: (i, 0)),
            pl.BlockSpec((tr, 128), lambda i: (i, 0)),
            pl.BlockSpec((tr, 128), lambda i: (i, 0)),
            pl.BlockSpec((tr, 128), lambda i: (i, 0)),
        ],
        out_shape=[
            jax.ShapeDtypeStruct((N, 64), jnp.float32),
            jax.ShapeDtypeStruct((N, 128), jnp.float32),
            jax.ShapeDtypeStruct((N, 128), jnp.float32),
            jax.ShapeDtypeStruct((N, 128), jnp.float32),
        ],
    )(f1, u1, mx1, mn1, st1, w2, b2r, we3, be3r)


def _final_body(f1_ref, f2_ref, f3_ref, u3_ref, mx_ref, mn_ref, st_ref,
                wout_ref, bout_ref, out_ref):
    s = st_ref[0:1, :]
    t = st_ref[1:2, :]
    hx = u3_ref[...] + jnp.where(s >= 0, mx_ref[...], mn_ref[...])
    f4 = jnp.maximum(s * hx + t, 0.0) + f3_ref[...]
    acc = jnp.dot(f1_ref[...], wout_ref[0:64, :], preferred_element_type=jnp.float32)
    acc = acc + jnp.dot(f2_ref[...], wout_ref[64:128, :], preferred_element_type=jnp.float32)
    acc = acc + jnp.dot(f3_ref[...], wout_ref[128:256, :], preferred_element_type=jnp.float32)
    acc = acc + jnp.dot(f4, wout_ref[256:384, :], preferred_element_type=jnp.float32)
    out_ref[...] = acc + bout_ref[0:1, :]


def _final(f1, f2, f3, u3, mx3, mn3, st3, wout, boutr):
    tr = 1000
    grid = (N // tr,)
    return pl.pallas_call(
        _final_body,
        grid=grid,
        in_specs=[
            pl.BlockSpec((tr, 64), lambda i: (i, 0)),
            pl.BlockSpec((tr, 64), lambda i: (i, 0)),
            pl.BlockSpec((tr, 128), lambda i: (i, 0)),
            pl.BlockSpec((tr, 128), lambda i: (i, 0)),
            pl.BlockSpec((tr, 128), lambda i: (i, 0)),
            pl.BlockSpec((tr, 128), lambda i: (i, 0)),
            pl.BlockSpec((8, 128), lambda i: (0, 0)),
            pl.BlockSpec((384, 128), lambda i: (0, 0)),
            pl.BlockSpec((8, 128), lambda i: (0, 0)),
        ],
        out_specs=pl.BlockSpec((tr, 128), lambda i: (i, 0)),
        out_shape=jax.ShapeDtypeStruct((N, 128), jnp.float32),
    )(f1, f2, f3, u3, mx3, mn3, st3, wout, boutr)


def kernel(point_features, point_coords, W0, b0, We1, be1, g1, bt1,
           W2, b2, We3, be3, g3, bt3, Wout, bout):
    pos = point_coords[:, 1:4]
    x = jnp.concatenate([pos, point_features], axis=1)

    posr = jnp.zeros((NPAD, 4), jnp.float32).at[:N, :3].set(pos)
    post = jnp.full((8, NPAD), 1.0e6, jnp.float32).at[0:3, :N].set(pos.T)
    idx = _knn(posr, post)
    idx_flat = idx.reshape(-1)

    def row8(b):
        return jnp.broadcast_to(b[None, :], (8, b.shape[0]))

    f1, u1, v1 = _mm1(x, W0, row8(b0), We1, row8(be1))

    v1p = jnp.zeros((NPAD, 128), jnp.float32).at[:N, :64].set(v1)
    s1, q1, mx1, mn1 = _sc_gather_reduce(v1p, idx_flat, 64)
    s1, q1, mx1, mn1 = s1[:N], q1[:N], mx1[:N], mn1[:N]
    st1 = _stats(u1, s1, q1, row8(g1), row8(bt1))

    f2, f3, u3, v3 = _mid(f1, u1, mx1, mn1, st1, W2, row8(b2), We3, row8(be3))

    v3p = jnp.zeros((NPAD, 128), jnp.float32).at[:N].set(v3)
    s3, q3, mx3, mn3 = _sc_gather_reduce(v3p, idx_flat, 128)
    s3, q3, mx3, mn3 = s3[:N], q3[:N], mx3[:N], mn3[:N]
    st3 = _stats(u3, s3, q3, row8(g3), row8(bt3))

    return _final(f1, f2, f3, u3, mx3, mn3, st3, Wout, row8(bout))


# speedup vs baseline: 8.4508x; 1.7674x over previous
import functools

import jax
import jax.numpy as jnp
from jax import lax
from jax.experimental import pallas as pl
from jax.experimental.pallas import tpu as pltpu
from jax.experimental.pallas import tpu_sc as plsc

N = 10000
K = 16
NPAD = 10240
T_KNN = 256
BIG = 3.0e38


def _cmp_full(av, ai, bv, bi):
    c = av <= bv
    return (jnp.minimum(av, bv), jnp.where(c, ai, bi),
            jnp.maximum(av, bv), jnp.where(c, bi, ai))


def _cmp_lo(av, ai, bv, bi):
    c = av <= bv
    return jnp.minimum(av, bv), jnp.where(c, ai, bi)


def _top3_merge(a, b):
    (a0v, a0i), (a1v, a1i), (a2v, a2i) = a
    (b0v, b0i), (b1v, b1i), (b2v, b2i) = b
    m0v, m0i, xv, xi = _cmp_full(a0v, a0i, b0v, b0i)
    yv, yi = _cmp_lo(a1v, a1i, b1v, b1i)
    m1v, m1i, hv, hi = _cmp_full(xv, xi, yv, yi)
    lv, li = _cmp_lo(a2v, a2i, b2v, b2i)
    m2v, m2i = _cmp_lo(hv, hi, lv, li)
    return [(m0v, m0i), (m1v, m1i), (m2v, m2i)]


def _knn_body(posr_ref, post_ref, idx_ref):
    t = posr_ref.shape[0]
    npad = post_ref.shape[1]
    nslab = 8
    w = npad // nslab
    d = None
    for c in range(3):
        a = posr_ref[:, c:c + 1]
        b = post_ref[c:c + 1, :]
        diff = a - b
        sq = diff * diff
        d = sq if d is None else d + sq

    slabs = [(d[:, k * w:(k + 1) * w],
              jnp.full((t, w), k, jnp.int32)) for k in range(nslab)]
    pairs = []
    for p in range(4):
        av, ai = slabs[2 * p]
        bv, bi = slabs[2 * p + 1]
        lov, loi, hiv, hii = _cmp_full(av, ai, bv, bi)
        pairs.append(((lov, loi), (hiv, hii)))

    def merge_pair(a, b):
        (a0v, a0i), (a1v, a1i) = a
        (b0v, b0i), (b1v, b1i) = b
        m0v, m0i, xv, xi = _cmp_full(a0v, a0i, b0v, b0i)
        yv, yi, zv, zi = _cmp_full(a1v, a1i, b1v, b1i)
        m1v, m1i, hv, hi = _cmp_full(xv, xi, yv, yi)
        m2v, m2i = _cmp_lo(hv, hi, zv, zi)
        return [(m0v, m0i), (m1v, m1i), (m2v, m2i)]

    t3a = merge_pair(pairs[0], pairs[1])
    t3b = merge_pair(pairs[2], pairs[3])
    top3 = _top3_merge(t3a, t3b)

    pos = lax.broadcasted_iota(jnp.int32, (t, w), 1)
    vs, ids = [], []
    for v, sl in top3:
        vs.append(v)
        ids.append(sl * w + pos)
    vcand = jnp.concatenate(vs, axis=1)
    icand = jnp.concatenate(ids, axis=1)

    cols = []
    for _ in range(K):
        m = jnp.min(vcand, axis=1, keepdims=True)
        cand = jnp.where(vcand == m, icand, npad)
        j = jnp.min(cand, axis=1, keepdims=True)
        cols.append(j)
        vcand = jnp.where(cand == j, BIG, vcand)
    idx_ref[...] = jnp.concatenate(cols, axis=1)


def _knn(posr, post):
    grid = (NPAD // T_KNN,)
    return pl.pallas_call(
        _knn_body,
        grid=grid,
        in_specs=[
            pl.BlockSpec((T_KNN, 4), lambda i: (i, 0)),
            pl.BlockSpec((8, NPAD), lambda i: (0, 0)),
        ],
        out_specs=pl.BlockSpec((T_KNN, K), lambda i: (i, 0)),
        out_shape=jax.ShapeDtypeStruct((NPAD, K), jnp.int32),
    )(posr, post)


def _mm1_body(x_ref, w0_ref, b0_ref, we_ref, be_ref, f1_ref, u_ref, v_ref):
    f1 = jnp.dot(x_ref[...], w0_ref[...], preferred_element_type=jnp.float32)
    f1 = f1 + b0_ref[0:1, :]
    c = w0_ref.shape[1]
    wa = we_ref[0:c, :]
    wb = we_ref[c:2 * c, :]
    u_ref[...] = jnp.dot(f1, wa - wb, preferred_element_type=jnp.float32) + be_ref[0:1, :]
    v_ref[...] = jnp.dot(f1, wb, preferred_element_type=jnp.float32)
    f1_ref[...] = f1


def _mm1(x, w0, b0r, we1, be1r):
    tr = 1000
    grid = (N // tr,)
    return pl.pallas_call(
        _mm1_body,
        grid=grid,
        in_specs=[
            pl.BlockSpec((tr, 64), lambda i: (i, 0)),
            pl.BlockSpec((64, 64), lambda i: (0, 0)),
            pl.BlockSpec((8, 64), lambda i: (0, 0)),
            pl.BlockSpec((128, 64), lambda i: (0, 0)),
            pl.BlockSpec((8, 64), lambda i: (0, 0)),
        ],
        out_specs=[
            pl.BlockSpec((tr, 64), lambda i: (i, 0)),
            pl.BlockSpec((tr, 64), lambda i: (i, 0)),
            pl.BlockSpec((tr, 64), lambda i: (i, 0)),
        ],
        out_shape=[jax.ShapeDtypeStruct((N, 64), jnp.float32)] * 3,
    )(x, w0, b0r, we1, be1r)


def _sc_gather_reduce(v_pad, idx_flat, c_dim):
    c_tab = v_pad.shape[1]
    nw = 32
    bpw = NPAD // nw
    nb = bpw // 8
    mesh = plsc.VectorSubcoreMesh(core_axis_name="c", subcore_axis_name="s")

    @functools.partial(
        pl.kernel,
        mesh=mesh,
        out_type=[jax.ShapeDtypeStruct((NPAD, c_dim), jnp.float32)] * 4,
        scratch_types=[
            pltpu.VMEM((128,), jnp.int32),
            pltpu.VMEM((128, c_tab), jnp.float32),
            pltpu.VMEM((8, c_dim), jnp.float32),
            pltpu.VMEM((8, c_dim), jnp.float32),
            pltpu.VMEM((8, c_dim), jnp.float32),
            pltpu.VMEM((8, c_dim), jnp.float32),
            pltpu.SemaphoreType.DMA,
        ],
    )
    def k(v_hbm, idx_hbm, s_hbm, q_hbm, mx_hbm, mn_hbm,
          idx_v, rows_v, s_v, q_v, mx_v, mn_v, sem):
        wid = lax.axis_index("s") * 2 + lax.axis_index("c")
        base = wid * bpw

        def batch(bi, carry):
            n0 = base + bi * 8
            pltpu.sync_copy(idx_hbm.at[pl.ds(n0 * K, 8 * K)], idx_v)
            pltpu.async_copy(v_hbm.at[idx_v], rows_v, sem).wait()

            def node(n, carry2):
                r0 = n * K
                for ch in range(c_dim // 16):
                    sl = pl.ds(ch * 16, 16)
                    x0 = rows_v[r0, sl]
                    sa = x0
                    qa = x0 * x0
                    mxa = x0
                    mna = x0
                    for r in range(1, K):
                        xr = rows_v[r0 + r, sl]
                        sa = sa + xr
                        qa = qa + xr * xr
                        mxa = jnp.maximum(mxa, xr)
                        mna = jnp.minimum(mna, xr)
                    s_v[n, sl] = sa
                    q_v[n, sl] = qa
                    mx_v[n, sl] = mxa
                    mn_v[n, sl] = mna
                return carry2

            lax.fori_loop(0, 8, node, 0)
            pltpu.sync_copy(s_v, s_hbm.at[pl.ds(n0, 8), :])
            pltpu.sync_copy(q_v, q_hbm.at[pl.ds(n0, 8), :])
            pltpu.sync_copy(mx_v, mx_hbm.at[pl.ds(n0, 8), :])
            pltpu.sync_copy(mn_v, mn_hbm.at[pl.ds(n0, 8), :])
            return carry

        lax.fori_loop(0, nb, batch, 0)

    return k(v_pad, idx_flat)


def _stats_body(u_ref, s_ref, q_ref, g_ref, bt_ref, st_ref):
    u = u_ref[...]
    s = s_ref[...]
    q = q_ref[...]
    su = jnp.sum(u, axis=0, keepdims=True)
    su2 = jnp.sum(u * u, axis=0, keepdims=True)
    sus = jnp.sum(u * s, axis=0, keepdims=True)
    ss = jnp.sum(s, axis=0, keepdims=True)
    sq = jnp.sum(q, axis=0, keepdims=True)
    denom = jnp.float32(N * K)
    mu = (K * su + ss) / denom
    eh2 = (K * su2 + 2.0 * sus + sq) / denom
    var = eh2 - mu * mu
    sc = g_ref[0:1, :] * lax.rsqrt(var + 1e-5)
    tc = bt_ref[0:1, :] - mu * sc
    st_ref[...] = jnp.concatenate([sc, tc, sc, tc, sc, tc, sc, tc], axis=0)


def _stats(u, s, q, gr, btr):
    c = u.shape[1]
    return pl.pallas_call(
        _stats_body,
        grid=(1,),
        in_specs=[
            pl.BlockSpec((N, c), lambda i: (0, 0)),
            pl.BlockSpec((N, c), lambda i: (0, 0)),
            pl.BlockSpec((N, c), lambda i: (0, 0)),
            pl.BlockSpec((8, c), lambda i: (0, 0)),
            pl.BlockSpec((8, c), lambda i: (0, 0)),
        ],
        out_specs=pl.BlockSpec((8, c), lambda i: (0, 0)),
        out_shape=jax.ShapeDtypeStruct((8, c), jnp.float32),
    )(u, s, q, gr, btr)


def _mid_body(f1_ref, u1_ref, mx_ref, mn_ref, st_ref, w2_ref, b2_ref,
              we3_ref, be3_ref, f2_ref, f3_ref, u3_ref, v3_ref):
    s = st_ref[0:1, :]
    t = st_ref[1:2, :]
    hx = u1_ref[...] + jnp.where(s >= 0, mx_ref[...], mn_ref[...])
    f2 = jnp.maximum(s * hx + t, 0.0) + f1_ref[...]
    f3 = jnp.dot(f2, w2_ref[...], preferred_element_type=jnp.float32) + b2_ref[0:1, :]
    c = w2_ref.shape[1]
    wa = we3_ref[0:c, :]
    wb = we3_ref[c:2 * c, :]
    u3_ref[...] = jnp.dot(f3, wa - wb, preferred_element_type=jnp.float32) + be3_ref[0:1, :]
    v3_ref[...] = jnp.dot(f3, wb, preferred_element_type=jnp.float32)
    f2_ref[...] = f2
    f3_ref[...] = f3


def _mid(f1, u1, mx1, mn1, st1, w2, b2r, we3, be3r):
    tr = 1000
    grid = (N // tr,)
    return pl.pallas_call(
        _mid_body,
        grid=grid,
        in_specs=[
            pl.BlockSpec((tr, 64), lambda i: (i, 0)),
            pl.BlockSpec((tr, 64), lambda i: (i, 0)),
            pl.BlockSpec((tr, 64), lambda i: (i, 0)),
            pl.BlockSpec((tr, 64), lambda i: (i, 0)),
            pl.BlockSpec((8, 64), lambda i: (0, 0)),
            pl.BlockSpec((64, 128), lambda i: (0, 0)),
            pl.BlockSpec((8, 128), lambda i: (0, 0)),
            pl.BlockSpec((256, 128), lambda i: (0, 0)),
            pl.BlockSpec((8, 128), lambda i: (0, 0)),
        ],
        out_specs=[
            pl.BlockSpec((tr, 64), lambda i: (i, 0)),
            pl.BlockSpec((tr, 128), lambda i: (i, 0)),
            pl.BlockSpec((tr, 128), lambda i: (i, 0)),
            pl.BlockSpec((tr, 128), lambda i: (i, 0)),
        ],
        out_shape=[
            jax.ShapeDtypeStruct((N, 64), jnp.float32),
            jax.ShapeDtypeStruct((N, 128), jnp.float32),
            jax.ShapeDtypeStruct((N, 128), jnp.float32),
            jax.ShapeDtypeStruct((N, 128), jnp.float32),
        ],
    )(f1, u1, mx1, mn1, st1, w2, b2r, we3, be3r)


def _final_body(f1_ref, f2_ref, f3_ref, u3_ref, mx_ref, mn_ref, st_ref,
                wout_ref, bout_ref, out_ref):
    s = st_ref[0:1, :]
    t = st_ref[1:2, :]
    hx = u3_ref[...] + jnp.where(s >= 0, mx_ref[...], mn_ref[...])
    f4 = jnp.maximum(s * hx + t, 0.0) + f3_ref[...]
    acc = jnp.dot(f1_ref[...], wout_ref[0:64, :], preferred_element_type=jnp.float32)
    acc = acc + jnp.dot(f2_ref[...], wout_ref[64:128, :], preferred_element_type=jnp.float32)
    acc = acc + jnp.dot(f3_ref[...], wout_ref[128:256, :], preferred_element_type=jnp.float32)
    acc = acc + jnp.dot(f4, wout_ref[256:384, :], preferred_element_type=jnp.float32)
    out_ref[...] = acc + bout_ref[0:1, :]


def _final(f1, f2, f3, u3, mx3, mn3, st3, wout, boutr):
    tr = 1000
    grid = (N // tr,)
    return pl.pallas_call(
        _final_body,
        grid=grid,
        in_specs=[
            pl.BlockSpec((tr, 64), lambda i: (i, 0)),
            pl.BlockSpec((tr, 64), lambda i: (i, 0)),
            pl.BlockSpec((tr, 128), lambda i: (i, 0)),
            pl.BlockSpec((tr, 128), lambda i: (i, 0)),
            pl.BlockSpec((tr, 128), lambda i: (i, 0)),
            pl.BlockSpec((tr, 128), lambda i: (i, 0)),
            pl.BlockSpec((8, 128), lambda i: (0, 0)),
            pl.BlockSpec((384, 128), lambda i: (0, 0)),
            pl.BlockSpec((8, 128), lambda i: (0, 0)),
        ],
        out_specs=pl.BlockSpec((tr, 128), lambda i: (i, 0)),
        out_shape=jax.ShapeDtypeStruct((N, 128), jnp.float32),
    )(f1, f2, f3, u3, mx3, mn3, st3, wout, boutr)


def kernel(point_features, point_coords, W0, b0, We1, be1, g1, bt1,
           W2, b2, We3, be3, g3, bt3, Wout, bout):
    pos = point_coords[:, 1:4]
    x = jnp.concatenate([pos, point_features], axis=1)

    posr = jnp.zeros((NPAD, 4), jnp.float32).at[:N, :3].set(pos)
    post = jnp.full((8, NPAD), 1.0e6, jnp.float32).at[0:3, :N].set(pos.T)
    idx = _knn(posr, post)
    idx_flat = idx.reshape(-1)

    def row8(b):
        return jnp.broadcast_to(b[None, :], (8, b.shape[0]))

    f1, u1, v1 = _mm1(x, W0, row8(b0), We1, row8(be1))

    v1p = jnp.zeros((NPAD, 128), jnp.float32).at[:N, :64].set(v1)
    s1, q1, mx1, mn1 = _sc_gather_reduce(v1p, idx_flat, 64)
    s1, q1, mx1, mn1 = s1[:N], q1[:N], mx1[:N], mn1[:N]
    st1 = _stats(u1, s1, q1, row8(g1), row8(bt1))

    f2, f3, u3, v3 = _mid(f1, u1, mx1, mn1, st1, W2, row8(b2), We3, row8(be3))

    v3p = jnp.zeros((NPAD, 128), jnp.float32).at[:N].set(v3)
    s3, q3, mx3, mn3 = _sc_gather_reduce(v3p, idx_flat, 128)
    s3, q3, mx3, mn3 = s3[:N], q3[:N], mx3[:N], mn3[:N]
    st3 = _stats(u3, s3, q3, row8(g3), row8(bt3))

    return _final(f1, f2, f3, u3, mx3, mn3, st3, Wout, row8(bout))


# speedup vs baseline: 10.7978x; 1.2777x over previous
import functools

import jax
import jax.numpy as jnp
from jax import lax
from jax.experimental import pallas as pl
from jax.experimental.pallas import tpu as pltpu
from jax.experimental.pallas import tpu_sc as plsc

N = 10000
K = 16
NPAD = 10240
T_KNN = 256
BIG = 3.0e38


def _cmp_full(av, ai, bv, bi):
    c = av <= bv
    return (jnp.minimum(av, bv), jnp.where(c, ai, bi),
            jnp.maximum(av, bv), jnp.where(c, bi, ai))


def _cmp_lo(av, ai, bv, bi):
    c = av <= bv
    return jnp.minimum(av, bv), jnp.where(c, ai, bi)


def _top3_merge(a, b):
    (a0v, a0i), (a1v, a1i), (a2v, a2i) = a
    (b0v, b0i), (b1v, b1i), (b2v, b2i) = b
    m0v, m0i, xv, xi = _cmp_full(a0v, a0i, b0v, b0i)
    yv, yi = _cmp_lo(a1v, a1i, b1v, b1i)
    m1v, m1i, hv, hi = _cmp_full(xv, xi, yv, yi)
    lv, li = _cmp_lo(a2v, a2i, b2v, b2i)
    m2v, m2i = _cmp_lo(hv, hi, lv, li)
    return [(m0v, m0i), (m1v, m1i), (m2v, m2i)]


def _knn_body(posr_ref, post_ref, idx_ref):
    t = posr_ref.shape[0]
    npad = post_ref.shape[1]
    nslab = 16
    w = npad // nslab
    d = None
    for c in range(3):
        a = posr_ref[:, c:c + 1]
        b = post_ref[c:c + 1, :]
        diff = a - b
        sq = diff * diff
        d = sq if d is None else d + sq

    slabs = [(d[:, k * w:(k + 1) * w],
              jnp.full((t, w), k, jnp.int32)) for k in range(nslab)]
    pairs = []
    for p in range(nslab // 2):
        av, ai = slabs[2 * p]
        bv, bi = slabs[2 * p + 1]
        lov, loi, hiv, hii = _cmp_full(av, ai, bv, bi)
        pairs.append(((lov, loi), (hiv, hii)))

    def merge_pair(a, b):
        (a0v, a0i), (a1v, a1i) = a
        (b0v, b0i), (b1v, b1i) = b
        m0v, m0i, xv, xi = _cmp_full(a0v, a0i, b0v, b0i)
        yv, yi, zv, zi = _cmp_full(a1v, a1i, b1v, b1i)
        m1v, m1i, hv, hi = _cmp_full(xv, xi, yv, yi)
        m2v, m2i = _cmp_lo(hv, hi, zv, zi)
        return [(m0v, m0i), (m1v, m1i), (m2v, m2i)]

    t3s = [merge_pair(pairs[2 * q], pairs[2 * q + 1])
           for q in range(nslab // 4)]
    while len(t3s) > 1:
        t3s = [_top3_merge(t3s[2 * q], t3s[2 * q + 1])
               for q in range(len(t3s) // 2)]
    top3 = t3s[0]

    pos = lax.broadcasted_iota(jnp.int32, (t, w), 1)
    vs, ids = [], []
    for v, sl in top3:
        vs.append(v)
        ids.append(sl * w + pos)
    vcand = jnp.concatenate(vs, axis=1)
    icand = jnp.concatenate(ids, axis=1)

    cols = []
    for _ in range(K):
        m = jnp.min(vcand, axis=1, keepdims=True)
        cand = jnp.where(vcand == m, icand, npad)
        j = jnp.min(cand, axis=1, keepdims=True)
        cols.append(j)
        vcand = jnp.where(cand == j, BIG, vcand)
    idx_ref[...] = jnp.concatenate(cols, axis=1)


def _knn(posr, post):
    grid = (NPAD // T_KNN,)
    return pl.pallas_call(
        _knn_body,
        grid=grid,
        in_specs=[
            pl.BlockSpec((T_KNN, 4), lambda i: (i, 0)),
            pl.BlockSpec((8, NPAD), lambda i: (0, 0)),
        ],
        out_specs=pl.BlockSpec((T_KNN, K), lambda i: (i, 0)),
        out_shape=jax.ShapeDtypeStruct((NPAD, K), jnp.int32),
    )(posr, post)


def _mm1_body(x_ref, w0_ref, b0_ref, we_ref, be_ref, f1_ref, u_ref, v_ref):
    f1 = jnp.dot(x_ref[...], w0_ref[...], preferred_element_type=jnp.float32)
    f1 = f1 + b0_ref[0:1, :]
    c = w0_ref.shape[1]
    wa = we_ref[0:c, :]
    wb = we_ref[c:2 * c, :]
    u_ref[...] = jnp.dot(f1, wa - wb, preferred_element_type=jnp.float32) + be_ref[0:1, :]
    v_ref[...] = jnp.dot(f1, wb, preferred_element_type=jnp.float32)
    f1_ref[...] = f1


def _mm1(x, w0, b0r, we1, be1r):
    tr = 1000
    grid = (N // tr,)
    return pl.pallas_call(
        _mm1_body,
        grid=grid,
        in_specs=[
            pl.BlockSpec((tr, 64), lambda i: (i, 0)),
            pl.BlockSpec((64, 64), lambda i: (0, 0)),
            pl.BlockSpec((8, 64), lambda i: (0, 0)),
            pl.BlockSpec((128, 64), lambda i: (0, 0)),
            pl.BlockSpec((8, 64), lambda i: (0, 0)),
        ],
        out_specs=[
            pl.BlockSpec((tr, 64), lambda i: (i, 0)),
            pl.BlockSpec((tr, 64), lambda i: (i, 0)),
            pl.BlockSpec((tr, 64), lambda i: (i, 0)),
        ],
        out_shape=[jax.ShapeDtypeStruct((N, 64), jnp.float32)] * 3,
    )(x, w0, b0r, we1, be1r)


def _sc_gather_reduce(v_pad, idx_flat, c_dim):
    c_tab = v_pad.shape[1]
    nw = 32
    bpw = NPAD // nw
    nb = bpw // 8
    mesh = plsc.VectorSubcoreMesh(core_axis_name="c", subcore_axis_name="s")

    @functools.partial(
        pl.kernel,
        mesh=mesh,
        out_type=[jax.ShapeDtypeStruct((NPAD, c_dim), jnp.float32)] * 4,
        scratch_types=[
            pltpu.VMEM((128,), jnp.int32),
            pltpu.VMEM((128, c_tab), jnp.float32),
            pltpu.VMEM((8, c_dim), jnp.float32),
            pltpu.VMEM((8, c_dim), jnp.float32),
            pltpu.VMEM((8, c_dim), jnp.float32),
            pltpu.VMEM((8, c_dim), jnp.float32),
            pltpu.SemaphoreType.DMA,
        ],
    )
    def k(v_hbm, idx_hbm, s_hbm, q_hbm, mx_hbm, mn_hbm,
          idx_v, rows_v, s_v, q_v, mx_v, mn_v, sem):
        wid = lax.axis_index("s") * 2 + lax.axis_index("c")
        base = wid * bpw

        def batch(bi, carry):
            n0 = base + bi * 8
            pltpu.sync_copy(idx_hbm.at[pl.ds(n0 * K, 8 * K)], idx_v)
            pltpu.async_copy(v_hbm.at[idx_v], rows_v, sem).wait()

            def node(n, carry2):
                r0 = n * K
                for ch in range(c_dim // 16):
                    sl = pl.ds(ch * 16, 16)
                    x0 = rows_v[r0, sl]
                    sa = x0
                    qa = x0 * x0
                    mxa = x0
                    mna = x0
                    for r in range(1, K):
                        xr = rows_v[r0 + r, sl]
                        sa = sa + xr
                        qa = qa + xr * xr
                        mxa = jnp.maximum(mxa, xr)
                        mna = jnp.minimum(mna, xr)
                    s_v[n, sl] = sa
                    q_v[n, sl] = qa
                    mx_v[n, sl] = mxa
                    mn_v[n, sl] = mna
                return carry2

            lax.fori_loop(0, 8, node, 0)
            pltpu.sync_copy(s_v, s_hbm.at[pl.ds(n0, 8), :])
            pltpu.sync_copy(q_v, q_hbm.at[pl.ds(n0, 8), :])
            pltpu.sync_copy(mx_v, mx_hbm.at[pl.ds(n0, 8), :])
            pltpu.sync_copy(mn_v, mn_hbm.at[pl.ds(n0, 8), :])
            return carry

        lax.fori_loop(0, nb, batch, 0)

    return k(v_pad, idx_flat)


def _stats_body(u_ref, s_ref, q_ref, g_ref, bt_ref, st_ref):
    u = u_ref[...]
    s = s_ref[...]
    q = q_ref[...]
    su = jnp.sum(u, axis=0, keepdims=True)
    su2 = jnp.sum(u * u, axis=0, keepdims=True)
    sus = jnp.sum(u * s, axis=0, keepdims=True)
    ss = jnp.sum(s, axis=0, keepdims=True)
    sq = jnp.sum(q, axis=0, keepdims=True)
    denom = jnp.float32(N * K)
    mu = (K * su + ss) / denom
    eh2 = (K * su2 + 2.0 * sus + sq) / denom
    var = eh2 - mu * mu
    sc = g_ref[0:1, :] * lax.rsqrt(var + 1e-5)
    tc = bt_ref[0:1, :] - mu * sc
    st_ref[...] = jnp.concatenate([sc, tc, sc, tc, sc, tc, sc, tc], axis=0)


def _stats(u, s, q, gr, btr):
    c = u.shape[1]
    return pl.pallas_call(
        _stats_body,
        grid=(1,),
        in_specs=[
            pl.BlockSpec((N, c), lambda i: (0, 0)),
            pl.BlockSpec((N, c), lambda i: (0, 0)),
            pl.BlockSpec((N, c), lambda i: (0, 0)),
            pl.BlockSpec((8, c), lambda i: (0, 0)),
            pl.BlockSpec((8, c), lambda i: (0, 0)),
        ],
        out_specs=pl.BlockSpec((8, c), lambda i: (0, 0)),
        out_shape=jax.ShapeDtypeStruct((8, c), jnp.float32),
    )(u, s, q, gr, btr)


def _mid_body(f1_ref, u1_ref, mx_ref, mn_ref, st_ref, w2_ref, b2_ref,
              we3_ref, be3_ref, f2_ref, f3_ref, u3_ref, v3_ref):
    s = st_ref[0:1, :]
    t = st_ref[1:2, :]
    hx = u1_ref[...] + jnp.where(s >= 0, mx_ref[...], mn_ref[...])
    f2 = jnp.maximum(s * hx + t, 0.0) + f1_ref[...]
    f3 = jnp.dot(f2, w2_ref[...], preferred_element_type=jnp.float32) + b2_ref[0:1, :]
    c = w2_ref.shape[1]
    wa = we3_ref[0:c, :]
    wb = we3_ref[c:2 * c, :]
    u3_ref[...] = jnp.dot(f3, wa - wb, preferred_element_type=jnp.float32) + be3_ref[0:1, :]
    v3_ref[...] = jnp.dot(f3, wb, preferred_element_type=jnp.float32)
    f2_ref[...] = f2
    f3_ref[...] = f3


def _mid(f1, u1, mx1, mn1, st1, w2, b2r, we3, be3r):
    tr = 1000
    grid = (N // tr,)
    return pl.pallas_call(
        _mid_body,
        grid=grid,
        in_specs=[
            pl.BlockSpec((tr, 64), lambda i: (i, 0)),
            pl.BlockSpec((tr, 64), lambda i: (i, 0)),
            pl.BlockSpec((tr, 64), lambda i: (i, 0)),
            pl.BlockSpec((tr, 64), lambda i: (i, 0)),
            pl.BlockSpec((8, 64), lambda i: (0, 0)),
            pl.BlockSpec((64, 128), lambda i: (0, 0)),
            pl.BlockSpec((8, 128), lambda i: (0, 0)),
            pl.BlockSpec((256, 128), lambda i: (0, 0)),
            pl.BlockSpec((8, 128), lambda i: (0, 0)),
        ],
        out_specs=[
            pl.BlockSpec((tr, 64), lambda i: (i, 0)),
            pl.BlockSpec((tr, 128), lambda i: (i, 0)),
            pl.BlockSpec((tr, 128), lambda i: (i, 0)),
            pl.BlockSpec((tr, 128), lambda i: (i, 0)),
        ],
        out_shape=[
            jax.ShapeDtypeStruct((N, 64), jnp.float32),
            jax.ShapeDtypeStruct((N, 128), jnp.float32),
            jax.ShapeDtypeStruct((N, 128), jnp.float32),
            jax.ShapeDtypeStruct((N, 128), jnp.float32),
        ],
    )(f1, u1, mx1, mn1, st1, w2, b2r, we3, be3r)


def _final_body(f1_ref, f2_ref, f3_ref, u3_ref, mx_ref, mn_ref, st_ref,
                wout_ref, bout_ref, out_ref):
    s = st_ref[0:1, :]
    t = st_ref[1:2, :]
    hx = u3_ref[...] + jnp.where(s >= 0, mx_ref[...], mn_ref[...])
    f4 = jnp.maximum(s * hx + t, 0.0) + f3_ref[...]
    acc = jnp.dot(f1_ref[...], wout_ref[0:64, :], preferred_element_type=jnp.float32)
    acc = acc + jnp.dot(f2_ref[...], wout_ref[64:128, :], preferred_element_type=jnp.float32)
    acc = acc + jnp.dot(f3_ref[...], wout_ref[128:256, :], preferred_element_type=jnp.float32)
    acc = acc + jnp.dot(f4, wout_ref[256:384, :], preferred_element_type=jnp.float32)
    out_ref[...] = acc + bout_ref[0:1, :]


def _final(f1, f2, f3, u3, mx3, mn3, st3, wout, boutr):
    tr = 1000
    grid = (N // tr,)
    return pl.pallas_call(
        _final_body,
        grid=grid,
        in_specs=[
            pl.BlockSpec((tr, 64), lambda i: (i, 0)),
            pl.BlockSpec((tr, 64), lambda i: (i, 0)),
            pl.BlockSpec((tr, 128), lambda i: (i, 0)),
            pl.BlockSpec((tr, 128), lambda i: (i, 0)),
            pl.BlockSpec((tr, 128), lambda i: (i, 0)),
            pl.BlockSpec((tr, 128), lambda i: (i, 0)),
            pl.BlockSpec((8, 128), lambda i: (0, 0)),
            pl.BlockSpec((384, 128), lambda i: (0, 0)),
            pl.BlockSpec((8, 128), lambda i: (0, 0)),
        ],
        out_specs=pl.BlockSpec((tr, 128), lambda i: (i, 0)),
        out_shape=jax.ShapeDtypeStruct((N, 128), jnp.float32),
    )(f1, f2, f3, u3, mx3, mn3, st3, wout, boutr)


def kernel(point_features, point_coords, W0, b0, We1, be1, g1, bt1,
           W2, b2, We3, be3, g3, bt3, Wout, bout):
    pos = point_coords[:, 1:4]
    x = jnp.concatenate([pos, point_features], axis=1)

    posr = jnp.zeros((NPAD, 4), jnp.float32).at[:N, :3].set(pos)
    post = jnp.full((8, NPAD), 1.0e6, jnp.float32).at[0:3, :N].set(pos.T)
    idx = _knn(posr, post)
    idx_flat = idx.reshape(-1)

    def row8(b):
        return jnp.broadcast_to(b[None, :], (8, b.shape[0]))

    f1, u1, v1 = _mm1(x, W0, row8(b0), We1, row8(be1))

    v1p = jnp.zeros((NPAD, 128), jnp.float32).at[:N, :64].set(v1)
    s1, q1, mx1, mn1 = _sc_gather_reduce(v1p, idx_flat, 64)
    s1, q1, mx1, mn1 = s1[:N], q1[:N], mx1[:N], mn1[:N]
    st1 = _stats(u1, s1, q1, row8(g1), row8(bt1))

    f2, f3, u3, v3 = _mid(f1, u1, mx1, mn1, st1, W2, row8(b2), We3, row8(be3))

    v3p = jnp.zeros((NPAD, 128), jnp.float32).at[:N].set(v3)
    s3, q3, mx3, mn3 = _sc_gather_reduce(v3p, idx_flat, 128)
    s3, q3, mx3, mn3 = s3[:N], q3[:N], mx3[:N], mn3[:N]
    st3 = _stats(u3, s3, q3, row8(g3), row8(bt3))

    return _final(f1, f2, f3, u3, mx3, mn3, st3, Wout, row8(bout))


# speedup vs baseline: 12.8035x; 1.1857x over previous
import functools

import jax
import jax.numpy as jnp
from jax import lax
from jax.experimental import pallas as pl
from jax.experimental.pallas import tpu as pltpu
from jax.experimental.pallas import tpu_sc as plsc

N = 10000
K = 16
NPAD = 10240
T_KNN = 256
BIG = 3.0e38


def _cmp_full(av, ai, bv, bi):
    c = av <= bv
    return (jnp.minimum(av, bv), jnp.where(c, ai, bi),
            jnp.maximum(av, bv), jnp.where(c, bi, ai))


def _cmp_lo(av, ai, bv, bi):
    c = av <= bv
    return jnp.minimum(av, bv), jnp.where(c, ai, bi)


def _top3_merge(a, b):
    (a0v, a0i), (a1v, a1i), (a2v, a2i) = a
    (b0v, b0i), (b1v, b1i), (b2v, b2i) = b
    m0v, m0i, xv, xi = _cmp_full(a0v, a0i, b0v, b0i)
    yv, yi = _cmp_lo(a1v, a1i, b1v, b1i)
    m1v, m1i, hv, hi = _cmp_full(xv, xi, yv, yi)
    lv, li = _cmp_lo(a2v, a2i, b2v, b2i)
    m2v, m2i = _cmp_lo(hv, hi, lv, li)
    return [(m0v, m0i), (m1v, m1i), (m2v, m2i)]


def _knn_body(posr_ref, post_ref, idx_ref):
    t = posr_ref.shape[0]
    npad = post_ref.shape[1]
    nslab = 16
    w = npad // nslab
    d = None
    for c in range(3):
        a = posr_ref[:, c:c + 1]
        b = post_ref[c:c + 1, :]
        diff = a - b
        sq = diff * diff
        d = sq if d is None else d + sq

    slabs = [(d[:, k * w:(k + 1) * w],
              jnp.full((t, w), k, jnp.int32)) for k in range(nslab)]
    pairs = []
    for p in range(nslab // 2):
        av, ai = slabs[2 * p]
        bv, bi = slabs[2 * p + 1]
        lov, loi, hiv, hii = _cmp_full(av, ai, bv, bi)
        pairs.append(((lov, loi), (hiv, hii)))

    def merge_pair(a, b):
        (a0v, a0i), (a1v, a1i) = a
        (b0v, b0i), (b1v, b1i) = b
        m0v, m0i, xv, xi = _cmp_full(a0v, a0i, b0v, b0i)
        yv, yi, zv, zi = _cmp_full(a1v, a1i, b1v, b1i)
        m1v, m1i, hv, hi = _cmp_full(xv, xi, yv, yi)
        m2v, m2i = _cmp_lo(hv, hi, zv, zi)
        return [(m0v, m0i), (m1v, m1i), (m2v, m2i)]

    t3s = [merge_pair(pairs[2 * q], pairs[2 * q + 1])
           for q in range(nslab // 4)]
    while len(t3s) > 1:
        t3s = [_top3_merge(t3s[2 * q], t3s[2 * q + 1])
               for q in range(len(t3s) // 2)]
    top3 = t3s[0]

    pos = lax.broadcasted_iota(jnp.int32, (t, w), 1)
    vs, ids = [], []
    for v, sl in top3:
        vs.append(v)
        ids.append(sl * w + pos)
    vcand = jnp.concatenate(vs, axis=1)
    icand = jnp.concatenate(ids, axis=1)

    cols = []
    for _ in range(K):
        m = jnp.min(vcand, axis=1, keepdims=True)
        cand = jnp.where(vcand == m, icand, npad)
        j = jnp.min(cand, axis=1, keepdims=True)
        cols.append(j)
        vcand = jnp.where(cand == j, BIG, vcand)
    idx_ref[...] = jnp.concatenate(cols, axis=1)


def _knn(posr, post):
    grid = (NPAD // T_KNN,)
    return pl.pallas_call(
        _knn_body,
        grid=grid,
        in_specs=[
            pl.BlockSpec((T_KNN, 4), lambda i: (i, 0)),
            pl.BlockSpec((8, NPAD), lambda i: (0, 0)),
        ],
        out_specs=pl.BlockSpec((T_KNN, K), lambda i: (i, 0)),
        out_shape=jax.ShapeDtypeStruct((NPAD, K), jnp.int32),
    )(posr, post)


def _mm1_body(x_ref, w0_ref, b0_ref, we_ref, be_ref, f1_ref, u_ref, v_ref):
    f1 = jnp.dot(x_ref[...], w0_ref[...], preferred_element_type=jnp.float32)
    f1 = f1 + b0_ref[0:1, :]
    c = w0_ref.shape[1]
    wa = we_ref[0:c, :]
    wb = we_ref[c:2 * c, :]
    u_ref[...] = jnp.dot(f1, wa - wb, preferred_element_type=jnp.float32) + be_ref[0:1, :]
    v = jnp.dot(f1, wb, preferred_element_type=jnp.float32)
    v_ref[...] = jnp.concatenate([v, jnp.zeros_like(v)], axis=1)
    f1_ref[...] = f1


def _mm1(x, w0, b0r, we1, be1r):
    tr = 1000
    grid = (N // tr,)
    return pl.pallas_call(
        _mm1_body,
        grid=grid,
        in_specs=[
            pl.BlockSpec((tr, 64), lambda i: (i, 0)),
            pl.BlockSpec((64, 64), lambda i: (0, 0)),
            pl.BlockSpec((8, 64), lambda i: (0, 0)),
            pl.BlockSpec((128, 64), lambda i: (0, 0)),
            pl.BlockSpec((8, 64), lambda i: (0, 0)),
        ],
        out_specs=[
            pl.BlockSpec((tr, 64), lambda i: (i, 0)),
            pl.BlockSpec((tr, 64), lambda i: (i, 0)),
            pl.BlockSpec((tr, 128), lambda i: (i, 0)),
        ],
        out_shape=[
            jax.ShapeDtypeStruct((N, 64), jnp.float32),
            jax.ShapeDtypeStruct((N, 64), jnp.float32),
            jax.ShapeDtypeStruct((NPAD, 128), jnp.float32),
        ],
    )(x, w0, b0r, we1, be1r)


def _sc_gather_reduce(v_pad, idx_flat, c_dim):
    c_tab = v_pad.shape[1]
    nw = 32
    bpw = NPAD // nw
    nb = bpw // 8
    mesh = plsc.VectorSubcoreMesh(core_axis_name="c", subcore_axis_name="s")

    @functools.partial(
        pl.kernel,
        mesh=mesh,
        out_type=jax.ShapeDtypeStruct((NPAD, 4 * c_dim), jnp.float32),
        scratch_types=[
            pltpu.VMEM((bpw * K,), jnp.int32),
            pltpu.VMEM((128, c_tab), jnp.float32),
            pltpu.VMEM((128, c_tab), jnp.float32),
            pltpu.VMEM((8, 4 * c_dim), jnp.float32),
            pltpu.SemaphoreType.DMA,
            pltpu.SemaphoreType.DMA,
        ],
    )
    def k(v_hbm, idx_hbm, out_hbm, idx_v, rows0, rows1, st_v, sem0, sem1):
        wid = lax.axis_index("s") * 2 + lax.axis_index("c")
        base = wid * bpw
        rows = (rows0, rows1)
        sems = (sem0, sem1)
        pltpu.sync_copy(idx_hbm.at[pl.ds(base * K, bpw * K)], idx_v)
        pltpu.async_copy(v_hbm.at[idx_v.at[pl.ds(0, 128)]], rows0, sem0)

        def pair(b2, carry):
            for ph in range(2):
                bi = b2 * 2 + ph
                nxt = bi + 1

                @pl.when(nxt < nb)
                def _():
                    pltpu.async_copy(
                        v_hbm.at[idx_v.at[pl.ds(nxt * 128, 128)]],
                        rows[1 - ph], sems[1 - ph])

                pltpu.make_async_copy(
                    v_hbm.at[idx_v.at[pl.ds(bi * 128, 128)]],
                    rows[ph], sems[ph]).wait()
                rv = rows[ph]

                def node(n, carry2):
                    r0 = n * K
                    for ch in range(c_dim // 16):
                        sl = pl.ds(ch * 16, 16)
                        x0 = rv[r0, sl]
                        sa = x0
                        qa = x0 * x0
                        mxa = x0
                        mna = x0
                        for r in range(1, K):
                            xr = rv[r0 + r, sl]
                            sa = sa + xr
                            qa = qa + xr * xr
                            mxa = jnp.maximum(mxa, xr)
                            mna = jnp.minimum(mna, xr)
                        st_v[n, sl] = sa
                        st_v[n, pl.ds(c_dim + ch * 16, 16)] = qa
                        st_v[n, pl.ds(2 * c_dim + ch * 16, 16)] = mxa
                        st_v[n, pl.ds(3 * c_dim + ch * 16, 16)] = mna
                    return carry2

                lax.fori_loop(0, 8, node, 0)
                pltpu.sync_copy(st_v, out_hbm.at[pl.ds(base + bi * 8, 8), :])
            return carry

        lax.fori_loop(0, nb // 2, pair, 0)

    return k(v_pad, idx_flat)


def _stats_body(u_ref, sq_ref, g_ref, bt_ref, st_ref):
    c = u_ref.shape[1]
    u = u_ref[...]
    s = sq_ref[:, 0:c]
    q = sq_ref[:, c:2 * c]
    su = jnp.sum(u, axis=0, keepdims=True)
    su2 = jnp.sum(u * u, axis=0, keepdims=True)
    sus = jnp.sum(u * s, axis=0, keepdims=True)
    ss = jnp.sum(s, axis=0, keepdims=True)
    sq = jnp.sum(q, axis=0, keepdims=True)
    denom = jnp.float32(N * K)
    mu = (K * su + ss) / denom
    eh2 = (K * su2 + 2.0 * sus + sq) / denom
    var = eh2 - mu * mu
    sc = g_ref[0:1, :] * lax.rsqrt(var + 1e-5)
    tc = bt_ref[0:1, :] - mu * sc
    st_ref[...] = jnp.concatenate([sc, tc, sc, tc, sc, tc, sc, tc], axis=0)


def _stats(u, sqmm, gr, btr):
    c = u.shape[1]
    return pl.pallas_call(
        _stats_body,
        grid=(1,),
        in_specs=[
            pl.BlockSpec((N, c), lambda i: (0, 0)),
            pl.BlockSpec((N, 2 * c), lambda i: (0, 0)),
            pl.BlockSpec((8, c), lambda i: (0, 0)),
            pl.BlockSpec((8, c), lambda i: (0, 0)),
        ],
        out_specs=pl.BlockSpec((8, c), lambda i: (0, 0)),
        out_shape=jax.ShapeDtypeStruct((8, c), jnp.float32),
    )(u, sqmm, gr, btr)


def _mid_body(f1_ref, u1_ref, sq_ref, st_ref, w2_ref, b2_ref,
              we3_ref, be3_ref, f2_ref, f3_ref, u3_ref, v3_ref):
    cc = u1_ref.shape[1]
    s = st_ref[0:1, :]
    t = st_ref[1:2, :]
    mx = sq_ref[:, 2 * cc:3 * cc]
    mn = sq_ref[:, 3 * cc:4 * cc]
    hx = u1_ref[...] + jnp.where(s >= 0, mx, mn)
    f2 = jnp.maximum(s * hx + t, 0.0) + f1_ref[...]
    f3 = jnp.dot(f2, w2_ref[...], preferred_element_type=jnp.float32) + b2_ref[0:1, :]
    c = w2_ref.shape[1]
    wa = we3_ref[0:c, :]
    wb = we3_ref[c:2 * c, :]
    u3_ref[...] = jnp.dot(f3, wa - wb, preferred_element_type=jnp.float32) + be3_ref[0:1, :]
    v3_ref[...] = jnp.dot(f3, wb, preferred_element_type=jnp.float32)
    f2_ref[...] = f2
    f3_ref[...] = f3


def _mid(f1, u1, sqmm1, st1, w2, b2r, we3, be3r):
    tr = 1000
    grid = (N // tr,)
    return pl.pallas_call(
        _mid_body,
        grid=grid,
        in_specs=[
            pl.BlockSpec((tr, 64), lambda i: (i, 0)),
            pl.BlockSpec((tr, 64), lambda i: (i, 0)),
            pl.BlockSpec((tr, 256), lambda i: (i, 0)),
            pl.BlockSpec((8, 64), lambda i: (0, 0)),
            pl.BlockSpec((64, 128), lambda i: (0, 0)),
            pl.BlockSpec((8, 128), lambda i: (0, 0)),
            pl.BlockSpec((256, 128), lambda i: (0, 0)),
            pl.BlockSpec((8, 128), lambda i: (0, 0)),
        ],
        out_specs=[
            pl.BlockSpec((tr, 64), lambda i: (i, 0)),
            pl.BlockSpec((tr, 128), lambda i: (i, 0)),
            pl.BlockSpec((tr, 128), lambda i: (i, 0)),
            pl.BlockSpec((tr, 128), lambda i: (i, 0)),
        ],
        out_shape=[
            jax.ShapeDtypeStruct((N, 64), jnp.float32),
            jax.ShapeDtypeStruct((N, 128), jnp.float32),
            jax.ShapeDtypeStruct((N, 128), jnp.float32),
            jax.ShapeDtypeStruct((NPAD, 128), jnp.float32),
        ],
    )(f1, u1, sqmm1, st1, w2, b2r, we3, be3r)


def _final_body(f1_ref, f2_ref, f3_ref, u3_ref, sq_ref, st_ref,
                wout_ref, bout_ref, out_ref):
    cc = u3_ref.shape[1]
    s = st_ref[0:1, :]
    t = st_ref[1:2, :]
    mx = sq_ref[:, 2 * cc:3 * cc]
    mn = sq_ref[:, 3 * cc:4 * cc]
    hx = u3_ref[...] + jnp.where(s >= 0, mx, mn)
    f4 = jnp.maximum(s * hx + t, 0.0) + f3_ref[...]
    acc = jnp.dot(f1_ref[...], wout_ref[0:64, :], preferred_element_type=jnp.float32)
    acc = acc + jnp.dot(f2_ref[...], wout_ref[64:128, :], preferred_element_type=jnp.float32)
    acc = acc + jnp.dot(f3_ref[...], wout_ref[128:256, :], preferred_element_type=jnp.float32)
    acc = acc + jnp.dot(f4, wout_ref[256:384, :], preferred_element_type=jnp.float32)
    out_ref[...] = acc + bout_ref[0:1, :]


def _final(f1, f2, f3, u3, sqmm3, st3, wout, boutr):
    tr = 1000
    grid = (N // tr,)
    return pl.pallas_call(
        _final_body,
        grid=grid,
        in_specs=[
            pl.BlockSpec((tr, 64), lambda i: (i, 0)),
            pl.BlockSpec((tr, 64), lambda i: (i, 0)),
            pl.BlockSpec((tr, 128), lambda i: (i, 0)),
            pl.BlockSpec((tr, 128), lambda i: (i, 0)),
            pl.BlockSpec((tr, 512), lambda i: (i, 0)),
            pl.BlockSpec((8, 128), lambda i: (0, 0)),
            pl.BlockSpec((384, 128), lambda i: (0, 0)),
            pl.BlockSpec((8, 128), lambda i: (0, 0)),
        ],
        out_specs=pl.BlockSpec((tr, 128), lambda i: (i, 0)),
        out_shape=jax.ShapeDtypeStruct((N, 128), jnp.float32),
    )(f1, f2, f3, u3, sqmm3, st3, wout, boutr)


def kernel(point_features, point_coords, W0, b0, We1, be1, g1, bt1,
           W2, b2, We3, be3, g3, bt3, Wout, bout):
    pos = point_coords[:, 1:4]
    x = jnp.concatenate([pos, point_features], axis=1)

    posr = jnp.zeros((NPAD, 4), jnp.float32).at[:N, :3].set(pos)
    post = jnp.full((8, NPAD), 1.0e6, jnp.float32).at[0:3, :N].set(pos.T)
    idx = _knn(posr, post)
    idx_flat = idx.reshape(-1)

    def row8(b):
        return jnp.broadcast_to(b[None, :], (8, b.shape[0]))

    f1, u1, v1 = _mm1(x, W0, row8(b0), We1, row8(be1))

    sqmm1 = _sc_gather_reduce(v1, idx_flat, 64)
    st1 = _stats(u1, sqmm1, row8(g1), row8(bt1))

    f2, f3, u3, v3 = _mid(f1, u1, sqmm1, st1, W2, row8(b2), We3, row8(be3))

    sqmm3 = _sc_gather_reduce(v3, idx_flat, 128)
    st3 = _stats(u3, sqmm3, row8(g3), row8(bt3))

    return _final(f1, f2, f3, u3, sqmm3, st3, Wout, row8(bout))


# speedup vs baseline: 13.0497x; 1.0192x over previous
import functools

import jax
import jax.numpy as jnp
from jax import lax
from jax.experimental import pallas as pl
from jax.experimental.pallas import tpu as pltpu
from jax.experimental.pallas import tpu_sc as plsc

N = 10000
K = 16
NPAD = 10240
T_KNN = 256
BIG = 3.0e38


def _cmp_full(av, ai, bv, bi):
    c = av <= bv
    return (jnp.minimum(av, bv), jnp.where(c, ai, bi),
            jnp.maximum(av, bv), jnp.where(c, bi, ai))


def _cmp_lo(av, ai, bv, bi):
    c = av <= bv
    return jnp.minimum(av, bv), jnp.where(c, ai, bi)


def _top3_merge(a, b):
    (a0v, a0i), (a1v, a1i), (a2v, a2i) = a
    (b0v, b0i), (b1v, b1i), (b2v, b2i) = b
    m0v, m0i, xv, xi = _cmp_full(a0v, a0i, b0v, b0i)
    yv, yi = _cmp_lo(a1v, a1i, b1v, b1i)
    m1v, m1i, hv, hi = _cmp_full(xv, xi, yv, yi)
    lv, li = _cmp_lo(a2v, a2i, b2v, b2i)
    m2v, m2i = _cmp_lo(hv, hi, lv, li)
    return [(m0v, m0i), (m1v, m1i), (m2v, m2i)]


def _knn_body(posr_ref, post_ref, idx_ref):
    t = posr_ref.shape[0]
    npad = post_ref.shape[1]
    nslab = 16
    w = npad // nslab
    d = None
    for c in range(3):
        a = posr_ref[:, c:c + 1]
        b = post_ref[c:c + 1, :]
        diff = a - b
        sq = diff * diff
        d = sq if d is None else d + sq

    slabs = [(d[:, k * w:(k + 1) * w],
              jnp.full((t, w), k, jnp.int32)) for k in range(nslab)]
    pairs = []
    for p in range(nslab // 2):
        av, ai = slabs[2 * p]
        bv, bi = slabs[2 * p + 1]
        lov, loi, hiv, hii = _cmp_full(av, ai, bv, bi)
        pairs.append(((lov, loi), (hiv, hii)))

    def merge_pair(a, b):
        (a0v, a0i), (a1v, a1i) = a
        (b0v, b0i), (b1v, b1i) = b
        m0v, m0i, xv, xi = _cmp_full(a0v, a0i, b0v, b0i)
        yv, yi, zv, zi = _cmp_full(a1v, a1i, b1v, b1i)
        m1v, m1i, hv, hi = _cmp_full(xv, xi, yv, yi)
        m2v, m2i = _cmp_lo(hv, hi, zv, zi)
        return [(m0v, m0i), (m1v, m1i), (m2v, m2i)]

    t3s = [merge_pair(pairs[2 * q], pairs[2 * q + 1])
           for q in range(nslab // 4)]
    while len(t3s) > 1:
        t3s = [_top3_merge(t3s[2 * q], t3s[2 * q + 1])
               for q in range(len(t3s) // 2)]
    top3 = t3s[0]

    pos = lax.broadcasted_iota(jnp.int32, (t, w), 1)
    vs, ids = [], []
    for v, sl in top3:
        vs.append(v)
        ids.append(sl * w + pos)
    vcand = jnp.concatenate(vs, axis=1)
    icand = jnp.concatenate(ids, axis=1)

    w2 = (3 * w) // 8
    s2 = [(vcand[:, k * w2:(k + 1) * w2], icand[:, k * w2:(k + 1) * w2])
          for k in range(8)]
    pairs2 = []
    for p in range(4):
        av, ai = s2[2 * p]
        bv, bi = s2[2 * p + 1]
        lov, loi, hiv, hii = _cmp_full(av, ai, bv, bi)
        pairs2.append(((lov, loi), (hiv, hii)))

    def merge22(a, b):
        (a0v, a0i), (a1v, a1i) = a
        (b0v, b0i), (b1v, b1i) = b
        r0v, r0i, xv, xi = _cmp_full(a0v, a0i, b0v, b0i)
        yv, yi, r3v, r3i = _cmp_full(a1v, a1i, b1v, b1i)
        r1v, r1i, r2v, r2i = _cmp_full(xv, xi, yv, yi)
        return [(r0v, r0i), (r1v, r1i), (r2v, r2i), (r3v, r3i)]

    sa = merge22(pairs2[0], pairs2[1])
    sb = merge22(pairs2[2], pairs2[3])
    vs2, is2 = [], []
    for i in range(4):
        lv, li = _cmp_lo(sa[i][0], sa[i][1], sb[3 - i][0], sb[3 - i][1])
        vs2.append(lv)
        is2.append(li)
    vcand = jnp.concatenate(vs2, axis=1)
    icand = jnp.concatenate(is2, axis=1)

    cols = []
    for _ in range(K):
        m = jnp.min(vcand, axis=1, keepdims=True)
        cand = jnp.where(vcand == m, icand, npad)
        j = jnp.min(cand, axis=1, keepdims=True)
        cols.append(j)
        vcand = jnp.where(cand == j, BIG, vcand)
    idx_ref[...] = jnp.concatenate(cols, axis=1)


def _knn(posr, post):
    grid = (NPAD // T_KNN,)
    return pl.pallas_call(
        _knn_body,
        grid=grid,
        in_specs=[
            pl.BlockSpec((T_KNN, 4), lambda i: (i, 0)),
            pl.BlockSpec((8, NPAD), lambda i: (0, 0)),
        ],
        out_specs=pl.BlockSpec((T_KNN, K), lambda i: (i, 0)),
        out_shape=jax.ShapeDtypeStruct((NPAD, K), jnp.int32),
    )(posr, post)


def _mm1_body(x_ref, w0_ref, b0_ref, we_ref, be_ref, f1_ref, u_ref, v_ref):
    f1 = jnp.dot(x_ref[...], w0_ref[...], preferred_element_type=jnp.float32)
    f1 = f1 + b0_ref[0:1, :]
    c = w0_ref.shape[1]
    wa = we_ref[0:c, :]
    wb = we_ref[c:2 * c, :]
    u_ref[...] = jnp.dot(f1, wa - wb, preferred_element_type=jnp.float32) + be_ref[0:1, :]
    v = jnp.dot(f1, wb, preferred_element_type=jnp.float32)
    v_ref[...] = jnp.concatenate([v, jnp.zeros_like(v)], axis=1)
    f1_ref[...] = f1


def _mm1(x, w0, b0r, we1, be1r):
    tr = 1000
    grid = (N // tr,)
    return pl.pallas_call(
        _mm1_body,
        grid=grid,
        in_specs=[
            pl.BlockSpec((tr, 64), lambda i: (i, 0)),
            pl.BlockSpec((64, 64), lambda i: (0, 0)),
            pl.BlockSpec((8, 64), lambda i: (0, 0)),
            pl.BlockSpec((128, 64), lambda i: (0, 0)),
            pl.BlockSpec((8, 64), lambda i: (0, 0)),
        ],
        out_specs=[
            pl.BlockSpec((tr, 64), lambda i: (i, 0)),
            pl.BlockSpec((tr, 64), lambda i: (i, 0)),
            pl.BlockSpec((tr, 128), lambda i: (i, 0)),
        ],
        out_shape=[
            jax.ShapeDtypeStruct((N, 64), jnp.float32),
            jax.ShapeDtypeStruct((N, 64), jnp.float32),
            jax.ShapeDtypeStruct((NPAD, 128), jnp.float32),
        ],
    )(x, w0, b0r, we1, be1r)


def _sc_gather_reduce(v_pad, idx_flat, c_dim):
    c_tab = v_pad.shape[1]
    nw = 32
    bpw = NPAD // nw
    nb = bpw // 8
    mesh = plsc.VectorSubcoreMesh(core_axis_name="c", subcore_axis_name="s")

    @functools.partial(
        pl.kernel,
        mesh=mesh,
        out_type=jax.ShapeDtypeStruct((NPAD, 4 * c_dim), jnp.float32),
        scratch_types=[
            pltpu.VMEM((bpw * K,), jnp.int32),
            pltpu.VMEM((128, c_tab), jnp.float32),
            pltpu.VMEM((128, c_tab), jnp.float32),
            pltpu.VMEM((8, 4 * c_dim), jnp.float32),
            pltpu.SemaphoreType.DMA,
            pltpu.SemaphoreType.DMA,
        ],
    )
    def k(v_hbm, idx_hbm, out_hbm, idx_v, rows0, rows1, st_v, sem0, sem1):
        wid = lax.axis_index("s") * 2 + lax.axis_index("c")
        base = wid * bpw
        rows = (rows0, rows1)
        sems = (sem0, sem1)
        pltpu.sync_copy(idx_hbm.at[pl.ds(base * K, bpw * K)], idx_v)
        pltpu.async_copy(v_hbm.at[idx_v.at[pl.ds(0, 128)]], rows0, sem0)

        def pair(b2, carry):
            for ph in range(2):
                bi = b2 * 2 + ph
                nxt = bi + 1

                @pl.when(nxt < nb)
                def _():
                    pltpu.async_copy(
                        v_hbm.at[idx_v.at[pl.ds(nxt * 128, 128)]],
                        rows[1 - ph], sems[1 - ph])

                pltpu.make_async_copy(
                    v_hbm.at[idx_v.at[pl.ds(bi * 128, 128)]],
                    rows[ph], sems[ph]).wait()
                rv = rows[ph]

                def node(n, carry2):
                    r0 = n * K
                    for ch in range(c_dim // 16):
                        sl = pl.ds(ch * 16, 16)
                        x0 = rv[r0, sl]
                        sa = x0
                        qa = x0 * x0
                        mxa = x0
                        mna = x0
                        for r in range(1, K):
                            xr = rv[r0 + r, sl]
                            sa = sa + xr
                            qa = qa + xr * xr
                            mxa = jnp.maximum(mxa, xr)
                            mna = jnp.minimum(mna, xr)
                        st_v[n, sl] = sa
                        st_v[n, pl.ds(c_dim + ch * 16, 16)] = qa
                        st_v[n, pl.ds(2 * c_dim + ch * 16, 16)] = mxa
                        st_v[n, pl.ds(3 * c_dim + ch * 16, 16)] = mna
                    return carry2

                lax.fori_loop(0, 8, node, 0)
                pltpu.sync_copy(st_v, out_hbm.at[pl.ds(base + bi * 8, 8), :])
            return carry

        lax.fori_loop(0, nb // 2, pair, 0)

    return k(v_pad, idx_flat)


def _stats_body(u_ref, sq_ref, g_ref, bt_ref, st_ref):
    c = u_ref.shape[1]
    u = u_ref[...]
    s = sq_ref[:, 0:c]
    q = sq_ref[:, c:2 * c]
    su = jnp.sum(u, axis=0, keepdims=True)
    su2 = jnp.sum(u * u, axis=0, keepdims=True)
    sus = jnp.sum(u * s, axis=0, keepdims=True)
    ss = jnp.sum(s, axis=0, keepdims=True)
    sq = jnp.sum(q, axis=0, keepdims=True)
    denom = jnp.float32(N * K)
    mu = (K * su + ss) / denom
    eh2 = (K * su2 + 2.0 * sus + sq) / denom
    var = eh2 - mu * mu
    sc = g_ref[0:1, :] * lax.rsqrt(var + 1e-5)
    tc = bt_ref[0:1, :] - mu * sc
    st_ref[...] = jnp.concatenate([sc, tc, sc, tc, sc, tc, sc, tc], axis=0)


def _stats(u, sqmm, gr, btr):
    c = u.shape[1]
    return pl.pallas_call(
        _stats_body,
        grid=(1,),
        in_specs=[
            pl.BlockSpec((N, c), lambda i: (0, 0)),
            pl.BlockSpec((N, 2 * c), lambda i: (0, 0)),
            pl.BlockSpec((8, c), lambda i: (0, 0)),
            pl.BlockSpec((8, c), lambda i: (0, 0)),
        ],
        out_specs=pl.BlockSpec((8, c), lambda i: (0, 0)),
        out_shape=jax.ShapeDtypeStruct((8, c), jnp.float32),
    )(u, sqmm, gr, btr)


def _mid_body(f1_ref, u1_ref, sq_ref, st_ref, w2_ref, b2_ref,
              we3_ref, be3_ref, f2_ref, f3_ref, u3_ref, v3_ref):
    cc = u1_ref.shape[1]
    s = st_ref[0:1, :]
    t = st_ref[1:2, :]
    mx = sq_ref[:, 2 * cc:3 * cc]
    mn = sq_ref[:, 3 * cc:4 * cc]
    hx = u1_ref[...] + jnp.where(s >= 0, mx, mn)
    f2 = jnp.maximum(s * hx + t, 0.0) + f1_ref[...]
    f3 = jnp.dot(f2, w2_ref[...], preferred_element_type=jnp.float32) + b2_ref[0:1, :]
    c = w2_ref.shape[1]
    wa = we3_ref[0:c, :]
    wb = we3_ref[c:2 * c, :]
    u3_ref[...] = jnp.dot(f3, wa - wb, preferred_element_type=jnp.float32) + be3_ref[0:1, :]
    v3_ref[...] = jnp.dot(f3, wb, preferred_element_type=jnp.float32)
    f2_ref[...] = f2
    f3_ref[...] = f3


def _mid(f1, u1, sqmm1, st1, w2, b2r, we3, be3r):
    tr = 1000
    grid = (N // tr,)
    return pl.pallas_call(
        _mid_body,
        grid=grid,
        in_specs=[
            pl.BlockSpec((tr, 64), lambda i: (i, 0)),
            pl.BlockSpec((tr, 64), lambda i: (i, 0)),
            pl.BlockSpec((tr, 256), lambda i: (i, 0)),
            pl.BlockSpec((8, 64), lambda i: (0, 0)),
            pl.BlockSpec((64, 128), lambda i: (0, 0)),
            pl.BlockSpec((8, 128), lambda i: (0, 0)),
            pl.BlockSpec((256, 128), lambda i: (0, 0)),
            pl.BlockSpec((8, 128), lambda i: (0, 0)),
        ],
        out_specs=[
            pl.BlockSpec((tr, 64), lambda i: (i, 0)),
            pl.BlockSpec((tr, 128), lambda i: (i, 0)),
            pl.BlockSpec((tr, 128), lambda i: (i, 0)),
            pl.BlockSpec((tr, 128), lambda i: (i, 0)),
        ],
        out_shape=[
            jax.ShapeDtypeStruct((N, 64), jnp.float32),
            jax.ShapeDtypeStruct((N, 128), jnp.float32),
            jax.ShapeDtypeStruct((N, 128), jnp.float32),
            jax.ShapeDtypeStruct((NPAD, 128), jnp.float32),
        ],
    )(f1, u1, sqmm1, st1, w2, b2r, we3, be3r)


def _final_body(f1_ref, f2_ref, f3_ref, u3_ref, sq_ref, st_ref,
                wout_ref, bout_ref, out_ref):
    cc = u3_ref.shape[1]
    s = st_ref[0:1, :]
    t = st_ref[1:2, :]
    mx = sq_ref[:, 2 * cc:3 * cc]
    mn = sq_ref[:, 3 * cc:4 * cc]
    hx = u3_ref[...] + jnp.where(s >= 0, mx, mn)
    f4 = jnp.maximum(s * hx + t, 0.0) + f3_ref[...]
    acc = jnp.dot(f1_ref[...], wout_ref[0:64, :], preferred_element_type=jnp.float32)
    acc = acc + jnp.dot(f2_ref[...], wout_ref[64:128, :], preferred_element_type=jnp.float32)
    acc = acc + jnp.dot(f3_ref[...], wout_ref[128:256, :], preferred_element_type=jnp.float32)
    acc = acc + jnp.dot(f4, wout_ref[256:384, :], preferred_element_type=jnp.float32)
    out_ref[...] = acc + bout_ref[0:1, :]


def _final(f1, f2, f3, u3, sqmm3, st3, wout, boutr):
    tr = 1000
    grid = (N // tr,)
    return pl.pallas_call(
        _final_body,
        grid=grid,
        in_specs=[
            pl.BlockSpec((tr, 64), lambda i: (i, 0)),
            pl.BlockSpec((tr, 64), lambda i: (i, 0)),
            pl.BlockSpec((tr, 128), lambda i: (i, 0)),
            pl.BlockSpec((tr, 128), lambda i: (i, 0)),
            pl.BlockSpec((tr, 512), lambda i: (i, 0)),
            pl.BlockSpec((8, 128), lambda i: (0, 0)),
            pl.BlockSpec((384, 128), lambda i: (0, 0)),
            pl.BlockSpec((8, 128), lambda i: (0, 0)),
        ],
        out_specs=pl.BlockSpec((tr, 128), lambda i: (i, 0)),
        out_shape=jax.ShapeDtypeStruct((N, 128), jnp.float32),
    )(f1, f2, f3, u3, sqmm3, st3, wout, boutr)


def kernel(point_features, point_coords, W0, b0, We1, be1, g1, bt1,
           W2, b2, We3, be3, g3, bt3, Wout, bout):
    pos = point_coords[:, 1:4]
    x = jnp.concatenate([pos, point_features], axis=1)

    posr = jnp.zeros((NPAD, 4), jnp.float32).at[:N, :3].set(pos)
    post = jnp.full((8, NPAD), 1.0e6, jnp.float32).at[0:3, :N].set(pos.T)
    idx = _knn(posr, post)
    idx_flat = idx.reshape(-1)

    def row8(b):
        return jnp.broadcast_to(b[None, :], (8, b.shape[0]))

    f1, u1, v1 = _mm1(x, W0, row8(b0), We1, row8(be1))

    sqmm1 = _sc_gather_reduce(v1, idx_flat, 64)
    st1 = _stats(u1, sqmm1, row8(g1), row8(bt1))

    f2, f3, u3, v3 = _mid(f1, u1, sqmm1, st1, W2, row8(b2), We3, row8(be3))

    sqmm3 = _sc_gather_reduce(v3, idx_flat, 128)
    st3 = _stats(u3, sqmm3, row8(g3), row8(bt3))

    return _final(f1, f2, f3, u3, sqmm3, st3, Wout, row8(bout))
